# baseline (speedup 1.0000x reference)
import dataclasses
import functools

import jax
import jax.numpy as jnp
from jax import lax
from jax.experimental import pallas as pl
from jax.experimental.pallas import tpu as pltpu
from jax.experimental.pallas import tpu_sc as plsc

_VOCAB = 1000
_EMBED = 128
_LANES = 16
_B = 1024
_L = 50
_NNEG = 5
_N = _B * _L
_NSAMP = _NNEG + 1

_NC = 2
_NS = 16
_NW = _NC * _NS
_TOK_W = _N // _NW
_CT = _LANES
_NCH = _TOK_W // _CT
_JP = _EMBED // 2
_JPAD = _JP + 1
_CTXW = _JP * _CT
_INW = _CTXW + _NSAMP * _CT


def _sc_scores(table_p, in_pack):
    mesh = plsc.VectorSubcoreMesh(core_axis_name="c", subcore_axis_name="s")
    cp = pltpu.CompilerParams()
    if "needs_layout_passes" in pltpu.CompilerParams.__dataclass_fields__:
        cp = dataclasses.replace(cp, needs_layout_passes=False)

    @functools.partial(
        pl.kernel,
        out_type=jax.ShapeDtypeStruct((_NW, _NCH, 128), jnp.float32),
        mesh=mesh,
        compiler_params=cp,
        scratch_types=[
            pltpu.VMEM((_VOCAB, _JPAD), jnp.int32),
            pltpu.VMEM((_INW,), jnp.int32),
            pltpu.VMEM((_INW,), jnp.int32),
            pltpu.VMEM((128,), jnp.float32),
            pltpu.VMEM((128,), jnp.float32),
            pltpu.SemaphoreType.DMA,
            pltpu.SemaphoreType.DMA,
            pltpu.SemaphoreType.DMA,
            pltpu.SemaphoreType.DMA,
            pltpu.SemaphoreType.DMA,
        ],
    )
    def score_kernel(
        table_hbm, in_hbm, out_hbm,
        w_v, in_v0, in_v1, sco_v0, sco_v1,
        wsem, csem_a, csem_b, ssem_a, ssem_b,
    ):
        wid = lax.axis_index("s") * _NC + lax.axis_index("c")

        pltpu.async_copy(table_hbm, w_v, wsem).wait()
        zeros = jnp.zeros((_LANES,), jnp.float32)
        for sco_v in (sco_v0, sco_v1):
            sco_v[pl.ds(96, _LANES)] = zeros
            sco_v[pl.ds(112, _LANES)] = zeros

        def in_copy(c, in_v, sem):
            return pltpu.make_async_copy(in_hbm.at[wid, c], in_v, sem)

        def sco_copy(c, sco_v, sem):
            return pltpu.make_async_copy(sco_v, out_hbm.at[wid, c], sem)

        def unpack(words):
            lo = plsc.bitcast(lax.shift_left(words, 16), jnp.float32)
            hi = plsc.bitcast(
                lax.bitwise_and(words, jnp.int32(-65536)), jnp.float32
            )
            return lo, hi

        def compute(in_v, sco_v):
            bases = [
                in_v[pl.ds(_CTXW + _LANES * k, _LANES)] for k in range(_NSAMP)
            ]
            accs = [jnp.zeros((_LANES,), jnp.float32) for _ in range(_NSAMP)]
            for j in range(_JP):
                ce, co = unpack(in_v[pl.ds(_LANES * j, _LANES)])
                col = jnp.full((_LANES,), j, jnp.int32)
                for k in range(_NSAMP):
                    wp = plsc.load_gather(w_v, [bases[k], col])
                    w0, w1 = unpack(wp)
                    accs[k] = accs[k] + w0 * ce + w1 * co
            sco_v[pl.ds(0, _LANES)] = accs[0]
            for k in range(1, _NSAMP):
                sco_v[pl.ds(_LANES * k, _LANES)] = -accs[k]

        in_copy(0, in_v0, csem_a).start()
        in_copy(1, in_v1, csem_b).start()
        in_copy(0, in_v0, csem_a).wait()
        compute(in_v0, sco_v0)
        sco_copy(0, sco_v0, ssem_a).start()
        in_copy(2, in_v0, csem_a).start()
        in_copy(1, in_v1, csem_b).wait()
        compute(in_v1, sco_v1)
        sco_copy(1, sco_v1, ssem_b).start()
        in_copy(3, in_v1, csem_b).start()

        @pl.loop(2, _NCH - 2, step=2)
        def _(c):
            in_copy(c, in_v0, csem_a).wait()
            sco_copy(c, sco_v0, ssem_a).wait()
            compute(in_v0, sco_v0)
            sco_copy(c, sco_v0, ssem_a).start()
            in_copy(c + 2, in_v0, csem_a).start()
            in_copy(c + 1, in_v1, csem_b).wait()
            sco_copy(c + 1, sco_v1, ssem_b).wait()
            compute(in_v1, sco_v1)
            sco_copy(c + 1, sco_v1, ssem_b).start()
            in_copy(c + 3, in_v1, csem_b).start()

        ct = _NCH - 2
        in_copy(ct, in_v0, csem_a).wait()
        sco_copy(ct, sco_v0, ssem_a).wait()
        compute(in_v0, sco_v0)
        sco_copy(ct, sco_v0, ssem_a).start()
        in_copy(ct + 1, in_v1, csem_b).wait()
        sco_copy(ct + 1, sco_v1, ssem_b).wait()
        compute(in_v1, sco_v1)
        sco_copy(ct + 1, sco_v1, ssem_b).start()
        sco_copy(ct, sco_v0, ssem_a).wait()
        sco_copy(ct + 1, sco_v1, ssem_b).wait()

    return score_kernel(table_p, in_pack)


def _logsig(x):
    return jnp.minimum(x, 0.0) - jnp.log1p(jnp.exp(-jnp.abs(x)))


def _tc_loss(scores):
    def body(s_ref, o_ref):
        x = s_ref[...]
        lane = lax.broadcasted_iota(jnp.int32, x.shape, 1)
        o_ref[0, 0] = -jnp.sum(jnp.where(lane < 96, _logsig(x), 0.0))

    out = pl.pallas_call(
        body,
        out_specs=pl.BlockSpec(memory_space=pltpu.SMEM),
        out_shape=jax.ShapeDtypeStruct((1, 1), jnp.float32),
    )(scores)
    return out[0, 0]


def kernel(sentence, context, W, neg_samples):
    table_p = lax.bitcast_convert_type(
        W.astype(jnp.bfloat16).reshape(_VOCAB, _JP, 2), jnp.int32
    )
    table_p = jnp.concatenate(
        [table_p, jnp.zeros((_VOCAB, _JPAD - _JP), jnp.int32)], axis=1
    )
    ctx_p = lax.bitcast_convert_type(
        context.astype(jnp.bfloat16)
        .reshape(_NW, _NCH, _CT, _JP, 2)
        .transpose(0, 1, 3, 2, 4),
        jnp.int32,
    ).reshape(_NW, _NCH, _CTXW)
    idx6 = jnp.concatenate(
        [sentence.reshape(1, _N), neg_samples.reshape(_N, _NNEG).T], axis=0
    )
    idxp = (
        idx6.reshape(_NSAMP, _NW, _NCH, _CT)
        .transpose(1, 2, 0, 3)
        .reshape(_NW, _NCH, _NSAMP * _CT)
        .astype(jnp.int32)
    )
    in_pack = jnp.concatenate([ctx_p, idxp], axis=2)
    scores = _sc_scores(table_p, in_pack)
    return _tc_loss(scores.reshape(_NW * _NCH, 128))


# speedup vs baseline: 1.2135x; 1.2135x over previous
import dataclasses
import functools

import jax
import jax.numpy as jnp
from jax import lax
from jax.experimental import pallas as pl
from jax.experimental.pallas import tpu as pltpu
from jax.experimental.pallas import tpu_sc as plsc

_VOCAB = 1000
_EMBED = 128
_LANES = 16
_B = 1024
_L = 50
_NNEG = 5
_N = _B * _L
_NSAMP = _NNEG + 1

_NC = 2
_NS = 16
_NW = _NC * _NS
_TOK_W = _N // _NW
_JP = _EMBED // 2
_GT = _LANES
_GW = _JP * _GT + _NSAMP * _GT
_GRP = 5
_CT = _GRP * _GT
_NCH = _TOK_W // _CT
_INW = _GRP * _GW
_SCW = _GRP * 128


def _sc_scores(table_p, in_pack):
    mesh = plsc.VectorSubcoreMesh(core_axis_name="c", subcore_axis_name="s")
    cp = pltpu.CompilerParams()
    if "needs_layout_passes" in pltpu.CompilerParams.__dataclass_fields__:
        cp = dataclasses.replace(cp, needs_layout_passes=False)

    @functools.partial(
        pl.kernel,
        out_type=jax.ShapeDtypeStruct((_NW, _NCH, _SCW), jnp.float32),
        mesh=mesh,
        compiler_params=cp,
        scratch_types=[
            pltpu.VMEM((_VOCAB * _JP,), jnp.int32),
            pltpu.VMEM((_INW,), jnp.int32),
            pltpu.VMEM((_INW,), jnp.int32),
            pltpu.VMEM((_SCW,), jnp.float32),
            pltpu.VMEM((_SCW,), jnp.float32),
            pltpu.SemaphoreType.DMA,
            pltpu.SemaphoreType.DMA,
            pltpu.SemaphoreType.DMA,
            pltpu.SemaphoreType.DMA,
            pltpu.SemaphoreType.DMA,
        ],
    )
    def score_kernel(
        table_hbm, in_hbm, out_hbm,
        w_v, in_v0, in_v1, sco_v0, sco_v1,
        wsem, csem_a, csem_b, ssem_a, ssem_b,
    ):
        wid = lax.axis_index("s") * _NC + lax.axis_index("c")

        pltpu.async_copy(table_hbm, w_v, wsem).wait()
        lane = lax.iota(jnp.int32, _LANES)
        masks = [lane == t for t in range(_GT)]
        zero_i = jnp.zeros((_LANES,), jnp.int32)
        zero_f = jnp.zeros((_LANES,), jnp.float32)

        def in_copy(c, in_v, sem):
            return pltpu.make_async_copy(in_hbm.at[wid, c], in_v, sem)

        def sco_copy(c, sco_v, sem):
            return pltpu.make_async_copy(sco_v, out_hbm.at[wid, c], sem)

        def unpack(words):
            lo = plsc.bitcast(lax.shift_left(words, 16), jnp.float32)
            hi = plsc.bitcast(
                lax.bitwise_and(words, jnp.int32(-65536)), jnp.float32
            )
            return lo, hi

        def compute(in_v, sco_v):
            @pl.loop(0, _GRP)
            def _(g):
                off = g * _GW
                soff = g * 128
                sco_v[pl.ds(soff + 96, _LANES)] = zero_f
                sco_v[pl.ds(soff + 112, _LANES)] = zero_f
                bases = [
                    in_v[pl.ds(off + _JP * _GT + _LANES * k, _LANES)]
                    for k in range(_NSAMP)
                ]
                @pl.loop(0, _GT)
                def _(t):
                    mask_t = lane == t
                    cvec = [
                        unpack(in_v[pl.ds(off + _JP * t + _LANES * m, _LANES)])
                        for m in range(4)
                    ]
                    for k in range(_NSAMP):
                        r = jnp.sum(jnp.where(mask_t, bases[k], zero_i))
                        rb = r * _JP
                        acc = None
                        for m in range(4):
                            wlo, whi = unpack(w_v[pl.ds(rb + _LANES * m, _LANES)])
                            clo, chi = cvec[m]
                            term = wlo * clo + whi * chi
                            acc = term if acc is None else acc + term
                        if k > 0:
                            acc = -acc
                        cs = plsc.cumsum(acc)
                        pos = (soff + _LANES * k) + t
                        plsc.store_scatter(
                            sco_v,
                            [jnp.full((_LANES,), pos, jnp.int32)],
                            cs,
                            mask=masks[15],
                        )

        in_copy(0, in_v0, csem_a).start()
        in_copy(1, in_v1, csem_b).start()
        in_copy(0, in_v0, csem_a).wait()
        compute(in_v0, sco_v0)
        sco_copy(0, sco_v0, ssem_a).start()
        in_copy(2, in_v0, csem_a).start()
        in_copy(1, in_v1, csem_b).wait()
        compute(in_v1, sco_v1)
        sco_copy(1, sco_v1, ssem_b).start()
        in_copy(3, in_v1, csem_b).start()

        @pl.loop(2, _NCH - 2, step=2)
        def _(c):
            in_copy(c, in_v0, csem_a).wait()
            sco_copy(c, sco_v0, ssem_a).wait()
            compute(in_v0, sco_v0)
            sco_copy(c, sco_v0, ssem_a).start()
            in_copy(c + 2, in_v0, csem_a).start()
            in_copy(c + 1, in_v1, csem_b).wait()
            sco_copy(c + 1, sco_v1, ssem_b).wait()
            compute(in_v1, sco_v1)
            sco_copy(c + 1, sco_v1, ssem_b).start()
            in_copy(c + 3, in_v1, csem_b).start()

        ct = _NCH - 2
        in_copy(ct, in_v0, csem_a).wait()
        sco_copy(ct, sco_v0, ssem_a).wait()
        compute(in_v0, sco_v0)
        sco_copy(ct, sco_v0, ssem_a).start()
        in_copy(ct + 1, in_v1, csem_b).wait()
        sco_copy(ct + 1, sco_v1, ssem_b).wait()
        compute(in_v1, sco_v1)
        sco_copy(ct + 1, sco_v1, ssem_b).start()
        sco_copy(ct, sco_v0, ssem_a).wait()
        sco_copy(ct + 1, sco_v1, ssem_b).wait()

    return score_kernel(table_p, in_pack)


def _logsig(x):
    return jnp.minimum(x, 0.0) - jnp.log1p(jnp.exp(-jnp.abs(x)))


def _tc_loss(scores):
    def body(s_ref, o_ref):
        x = s_ref[...]
        lane = lax.broadcasted_iota(jnp.int32, x.shape, 1)
        o_ref[0, 0] = -jnp.sum(jnp.where(lane < 96, _logsig(x), 0.0))

    out = pl.pallas_call(
        body,
        out_specs=pl.BlockSpec(memory_space=pltpu.SMEM),
        out_shape=jax.ShapeDtypeStruct((1, 1), jnp.float32),
    )(scores)
    return out[0, 0]


def kernel(sentence, context, W, neg_samples):
    table_p = lax.bitcast_convert_type(
        W.astype(jnp.bfloat16).reshape(_VOCAB, _JP, 2), jnp.int32
    ).reshape(_VOCAB * _JP)
    ctx_p = lax.bitcast_convert_type(
        context.astype(jnp.bfloat16).reshape(_NW, _TOK_W // _GT, _GT, _JP, 2),
        jnp.int32,
    ).reshape(_NW, _TOK_W // _GT, _JP * _GT)
    idx6 = jnp.concatenate(
        [sentence.reshape(1, _N), neg_samples.reshape(_N, _NNEG).T], axis=0
    )
    idxp = (
        idx6.reshape(_NSAMP, _NW, _TOK_W // _GT, _GT)
        .transpose(1, 2, 0, 3)
        .reshape(_NW, _TOK_W // _GT, _NSAMP * _GT)
        .astype(jnp.int32)
    )
    in_pack = jnp.concatenate([ctx_p, idxp], axis=2).reshape(_NW, _NCH, _INW)
    scores = _sc_scores(table_p, in_pack)
    return _tc_loss(scores.reshape(_N * _NSAMP * 4 // 3 // 128, 128))
